# baseline (speedup 1.0000x reference)
import functools

import jax
import jax.numpy as jnp
from jax import lax
from jax.experimental import pallas as pl
from jax.experimental.pallas import tpu as pltpu
from jax.experimental.pallas import tpu_sc as plsc

NC, NS, LANES = 2, 16, 16
NW = NC * NS
D = 64
CHUNK = 128
SUB = 128
NBUF = 2


def _sc_dot_gather(n_total):
    per_w = n_total // NW
    n_chunks = per_w // CHUNK
    n_outer = n_chunks // NBUF
    mesh = plsc.VectorSubcoreMesh(core_axis_name="c", subcore_axis_name="s")

    @functools.partial(
        pl.kernel,
        out_type=jax.ShapeDtypeStruct((n_total,), jnp.float32),
        mesh=mesh,
        compiler_params=pltpu.CompilerParams(use_tc_tiling_on_sc=False),
        scratch_types=[
            pltpu.VMEM((per_w,), jnp.int32),
            pltpu.VMEM((per_w,), jnp.int32),
            pltpu.VMEM((CHUNK, D), jnp.float32),
            pltpu.VMEM((CHUNK, D), jnp.float32),
            pltpu.VMEM((CHUNK, D), jnp.float32),
            pltpu.VMEM((CHUNK, D), jnp.float32),
            pltpu.VMEM((CHUNK,), jnp.float32),
            pltpu.VMEM((CHUNK,), jnp.float32),
            pltpu.SemaphoreType.DMA,
            pltpu.SemaphoreType.DMA,
            pltpu.SemaphoreType.DMA,
            pltpu.SemaphoreType.DMA,
        ],
    )
    def sc_kernel(uid_hbm, mid_hbm, utab_hbm, mtab_hbm, out_hbm,
                  idx_u, idx_m, ru0, ru1, rm0, rm1, ob0, ob1,
                  sg0, sg1, so0, so1):
        wid = lax.axis_index("s") * NC + lax.axis_index("c")
        w0 = wid * per_w
        rows_u, rows_m = [ru0, ru1], [rm0, rm1]
        out_buf = [ob0, ob1]
        sgs, sos = [sg0, sg1], [so0, so1]

        pltpu.sync_copy(uid_hbm.at[pl.ds(w0, per_w)], idx_u)
        pltpu.sync_copy(mid_hbm.at[pl.ds(w0, per_w)], idx_m)

        def gathers(chunk, b):
            descs = []
            for j in range(CHUNK // SUB):
                s = pl.ds(chunk * CHUNK + j * SUB, SUB)
                d = pl.ds(j * SUB, SUB)
                descs.append(pltpu.make_async_copy(
                    utab_hbm.at[idx_u.at[s]], rows_u[b].at[d], sgs[b]))
                descs.append(pltpu.make_async_copy(
                    mtab_hbm.at[idx_m.at[s]], rows_m[b].at[d], sgs[b]))
            return descs

        def out_copy(chunk, b):
            return pltpu.make_async_copy(
                out_buf[b], out_hbm.at[pl.ds(w0 + chunk * CHUNK, CHUNK)],
                sos[b])

        lane = lax.iota(jnp.int32, LANES)
        xor_perms = [lane ^ sh for sh in (8, 4, 2, 1)]
        lo_half = lane < (LANES // 2)
        pair_masks = [(lane == a) | (lane == a + 8) for a in range(LANES // 2)]

        def compute(b):
            ru, rm, ob = rows_u[b], rows_m[b], out_buf[b]

            @functools.partial(lax.fori_loop, 0, CHUNK // LANES, init_val=None)
            def _(g, _carry):
                rbase = g * LANES
                prods = [ru[rbase + ri, pl.ds(0, LANES)]
                         * rm[rbase + ri, pl.ds(0, LANES)]
                         for ri in range(LANES)]
                for k in range(1, D // LANES):
                    s = pl.ds(k * LANES, LANES)
                    for ri in range(LANES):
                        r = rbase + ri
                        prods[ri] = prods[ri] + ru[r, s] * rm[r, s]
                grp = jnp.zeros((LANES,), jnp.float32)
                for a in range(LANES // 2):
                    pa, pb = prods[a], prods[a + 8]
                    ca = jnp.where(lo_half, pa + pa[xor_perms[0]],
                                   pb + pb[xor_perms[0]])
                    for p in xor_perms[1:]:
                        ca = ca + ca[p]
                    grp = jnp.where(pair_masks[a], ca, grp)
                ob[pl.ds(rbase, LANES)] = grp

        for cp in gathers(0, 0):
            cp.start()

        def outer(c2, carry):
            for b in range(NBUF):
                chunk = c2 * NBUF + b
                nxt_chunk = chunk + 1

                @pl.when(nxt_chunk < n_chunks)
                def _():
                    for cp in gathers(nxt_chunk, (b + 1) % NBUF):
                        cp.start()

                pltpu.make_async_copy(
                    utab_hbm.at[idx_u.at[pl.ds(0, SUB)]], rows_u[b],
                    sgs[b]).wait()
                pltpu.make_async_copy(
                    mtab_hbm.at[idx_m.at[pl.ds(0, SUB)]], rows_m[b],
                    sgs[b]).wait()

                @pl.when(chunk >= NBUF)
                def _():
                    out_copy(chunk - NBUF, b).wait()

                compute(b)
                out_copy(chunk, b).start()
            return carry

        lax.fori_loop(0, n_outer, outer, 0)
        for b in range(NBUF):
            out_copy(n_chunks - NBUF + b, b).wait()

    return sc_kernel


def kernel(user_id, movie_title, user_table, movie_table):
    b, l = user_id.shape
    n = b * l
    uid = user_id.reshape(n).astype(jnp.int32)
    mid = movie_title.reshape(n).astype(jnp.int32)
    out = _sc_dot_gather(n)(uid, mid, user_table, movie_table)
    return out.reshape(b, l)


# speedup vs baseline: 1.0383x; 1.0383x over previous
import functools

import jax
import jax.numpy as jnp
from jax import lax
from jax.experimental import pallas as pl
from jax.experimental.pallas import tpu as pltpu
from jax.experimental.pallas import tpu_sc as plsc

NC, NS, LANES = 2, 16, 16
NW = NC * NS
D = 64
CHUNK = 256
SUB = 128
NBUF = 2


def _sc_dot_gather(n_total):
    per_w = n_total // NW
    n_chunks = per_w // CHUNK
    n_outer = n_chunks // NBUF
    mesh = plsc.VectorSubcoreMesh(core_axis_name="c", subcore_axis_name="s")

    @functools.partial(
        pl.kernel,
        out_type=jax.ShapeDtypeStruct((n_total,), jnp.float32),
        mesh=mesh,
        compiler_params=pltpu.CompilerParams(use_tc_tiling_on_sc=False),
        scratch_types=[
            pltpu.VMEM((per_w,), jnp.int32),
            pltpu.VMEM((per_w,), jnp.int32),
            pltpu.VMEM((CHUNK, D), jnp.float32),
            pltpu.VMEM((CHUNK, D), jnp.float32),
            pltpu.VMEM((CHUNK, D), jnp.float32),
            pltpu.VMEM((CHUNK, D), jnp.float32),
            pltpu.VMEM((CHUNK,), jnp.float32),
            pltpu.VMEM((CHUNK,), jnp.float32),
            pltpu.SemaphoreType.DMA,
            pltpu.SemaphoreType.DMA,
            pltpu.SemaphoreType.DMA,
            pltpu.SemaphoreType.DMA,
        ],
    )
    def sc_kernel(uid_hbm, mid_hbm, utab_hbm, mtab_hbm, out_hbm,
                  idx_u, idx_m, ru0, ru1, rm0, rm1, ob0, ob1,
                  sg0, sg1, so0, so1):
        wid = lax.axis_index("s") * NC + lax.axis_index("c")
        w0 = wid * per_w
        rows_u, rows_m = [ru0, ru1], [rm0, rm1]
        out_buf = [ob0, ob1]
        sgs, sos = [sg0, sg1], [so0, so1]

        pltpu.sync_copy(uid_hbm.at[pl.ds(w0, per_w)], idx_u)
        pltpu.sync_copy(mid_hbm.at[pl.ds(w0, per_w)], idx_m)

        def gathers(chunk, b):
            descs = []
            for j in range(CHUNK // SUB):
                s = pl.ds(chunk * CHUNK + j * SUB, SUB)
                d = pl.ds(j * SUB, SUB)
                descs.append(pltpu.make_async_copy(
                    utab_hbm.at[idx_u.at[s]], rows_u[b].at[d], sgs[b]))
                descs.append(pltpu.make_async_copy(
                    mtab_hbm.at[idx_m.at[s]], rows_m[b].at[d], sgs[b]))
            return descs

        def out_copy(chunk, b):
            return pltpu.make_async_copy(
                out_buf[b], out_hbm.at[pl.ds(w0 + chunk * CHUNK, CHUNK)],
                sos[b])

        lane = lax.iota(jnp.int32, LANES)
        xor_perms = [lane ^ sh for sh in (8, 4, 2, 1)]
        lo_half = lane < (LANES // 2)
        pair_masks = [(lane == a) | (lane == a + 8) for a in range(LANES // 2)]

        def compute(b):
            ru, rm, ob = rows_u[b], rows_m[b], out_buf[b]

            @functools.partial(lax.fori_loop, 0, CHUNK // LANES, init_val=None)
            def _(g, _carry):
                rbase = g * LANES
                prods = [ru[rbase + ri, pl.ds(0, LANES)]
                         * rm[rbase + ri, pl.ds(0, LANES)]
                         for ri in range(LANES)]
                for k in range(1, D // LANES):
                    s = pl.ds(k * LANES, LANES)
                    for ri in range(LANES):
                        r = rbase + ri
                        prods[ri] = prods[ri] + ru[r, s] * rm[r, s]
                grp = jnp.zeros((LANES,), jnp.float32)
                for a in range(LANES // 2):
                    pa, pb = prods[a], prods[a + 8]
                    ca = jnp.where(lo_half, pa + pa[xor_perms[0]],
                                   pb + pb[xor_perms[0]])
                    for p in xor_perms[1:]:
                        ca = ca + ca[p]
                    grp = jnp.where(pair_masks[a], ca, grp)
                ob[pl.ds(rbase, LANES)] = grp

        for cp in gathers(0, 0):
            cp.start()

        def outer(c2, carry):
            for b in range(NBUF):
                chunk = c2 * NBUF + b
                nxt_chunk = chunk + 1

                @pl.when(nxt_chunk < n_chunks)
                def _():
                    for cp in gathers(nxt_chunk, (b + 1) % NBUF):
                        cp.start()

                pltpu.make_async_copy(
                    utab_hbm.at[idx_u.at[pl.ds(0, SUB)]], rows_u[b],
                    sgs[b]).wait()
                pltpu.make_async_copy(
                    mtab_hbm.at[idx_m.at[pl.ds(0, SUB)]], rows_m[b],
                    sgs[b]).wait()

                @pl.when(chunk >= NBUF)
                def _():
                    out_copy(chunk - NBUF, b).wait()

                compute(b)
                out_copy(chunk, b).start()
            return carry

        lax.fori_loop(0, n_outer, outer, 0)
        for b in range(NBUF):
            out_copy(n_chunks - NBUF + b, b).wait()

    return sc_kernel


def kernel(user_id, movie_title, user_table, movie_table):
    b, l = user_id.shape
    n = b * l
    uid = user_id.reshape(n).astype(jnp.int32)
    mid = movie_title.reshape(n).astype(jnp.int32)
    out = _sc_dot_gather(n)(uid, mid, user_table, movie_table)
    return out.reshape(b, l)
